# baseline (speedup 1.0000x reference)
import functools

import jax
import jax.numpy as jnp
from jax.experimental import pallas as pl
from jax.experimental.pallas import tpu as pltpu
from jax.experimental.pallas import tpu_sc as plsc

B = 64
SEQ = 96
NN = 325
E = 2600
OUT = 24
HEADS = 4
H1 = 64
H2 = 128
NT = 8


_SC_NW = 16
_SC_CHUNK = 184
_SC_NIDX = _SC_NW * _SC_CHUNK
_SC_TOTAL = 105728
_SC_ZCHUNK = _SC_TOTAL // _SC_NW
_SC_PAD_IDX = 105626


def _make_count_sc():
    mesh = plsc.VectorSubcoreMesh(core_axis_name="c", subcore_axis_name="s",
                                  num_cores=1)

    @functools.partial(
        pl.kernel, mesh=mesh,
        out_type=jax.ShapeDtypeStruct((_SC_TOTAL,), jnp.float32),
        scratch_types=[
            pltpu.VMEM((_SC_CHUNK,), jnp.int32),
            pltpu.VMEM((_SC_CHUNK,), jnp.float32),
            pltpu.VMEM((_SC_ZCHUNK,), jnp.float32),
            pltpu.VMEM_SHARED((_SC_TOTAL,), jnp.float32),
        ],
    )
    def count_sc(idx_hbm, zeros_hbm, ones_hbm, out_hbm, idx_v, ones_v,
                 zbuf_v, shared):
        wid = jax.lax.axis_index("s")
        zb = wid * _SC_ZCHUNK
        pltpu.sync_copy(zeros_hbm.at[pl.ds(zb, _SC_ZCHUNK)], zbuf_v)
        pltpu.sync_copy(zbuf_v, shared.at[pl.ds(zb, _SC_ZCHUNK)])
        plsc.subcore_barrier()
        eb = wid * _SC_CHUNK
        pltpu.sync_copy(idx_hbm.at[pl.ds(eb, _SC_CHUNK)], idx_v)
        pltpu.sync_copy(ones_hbm.at[pl.ds(eb, _SC_CHUNK)], ones_v)
        pltpu.sync_copy(ones_v, shared.at[idx_v], add=True)
        plsc.subcore_barrier()
        pltpu.sync_copy(shared.at[pl.ds(zb, _SC_ZCHUNK)], zbuf_v)
        pltpu.sync_copy(zbuf_v, out_hbm.at[pl.ds(zb, _SC_ZCHUNK)])

    return count_sc


def _gat_layer(h, as_bd, ad_bd, cmat, oc):
    als_t = jax.lax.dot_general(as_bd, h, (((0,), (1,)), ((), ())),
                                preferred_element_type=jnp.float32)
    ald = jax.lax.dot_general(h, ad_bd, (((1,), (0,)), ((), ())),
                              preferred_element_type=jnp.float32)
    ones_col = jnp.ones((NN, 1), dtype=jnp.float32)
    ones_row = jnp.ones((1, NN), dtype=jnp.float32)
    acc = jnp.zeros((NN, oc), dtype=jnp.float32)
    for k in range(HEADS):
        row = als_t[k:k + 1, :]
        col = ald[:, k:k + 1]
        t = col + jnp.max(row)
        bd = jnp.maximum(t, 0.2 * t)
        e = col + row
        e = jnp.maximum(e, 0.2 * e)
        p = cmat * jnp.exp(e - bd)
        den = p.sum(axis=1, keepdims=True)
        rden = 1.0 / (den + 1e-16)
        hh = h[:, k * oc:(k + 1) * oc]
        num = jax.lax.dot_general(p, hh, (((1,), (0,)), ((), ())),
                                  preferred_element_type=jnp.float32)
        acc = acc + num * rden
    return acc * (1.0 / HEADS)


def _gat_body(x_ref, w1_ref, as1_ref, ad1_ref, b1_ref, w2_ref, as2_ref,
              ad2_ref, b2_ref, c_ref, out_ref):
    xb = x_ref[0]
    cmat = c_ref[...]

    h = jax.lax.dot_general(xb, w1_ref[...], (((0,), (0,)), ((), ())),
                            preferred_element_type=jnp.float32)
    o1 = _gat_layer(h, as1_ref[...], ad1_ref[...], cmat, H1) + b1_ref[...]
    o1 = jnp.where(o1 > 0, o1, jnp.exp(jnp.minimum(o1, 0.0)) - 1.0)

    h = jax.lax.dot_general(o1, w2_ref[...], (((1,), (0,)), ((), ())),
                            preferred_element_type=jnp.float32)
    out_ref[0] = _gat_layer(h, as2_ref[...], ad2_ref[...], cmat, H2) \
        + b2_ref[...]


def _mlp_body(g_ref, w1_ref, b1_ref, w2_ref, b2_ref, out_ref):
    for i in range(NT):
        g = g_ref[:, i, :]
        t = jax.lax.dot_general(g, w1_ref[i], (((1,), (0,)), ((), ())),
                                preferred_element_type=jnp.float32)
        t = jnp.maximum(t + b1_ref[i], 0.0)
        o = jax.lax.dot_general(t, w2_ref[i], (((1,), (0,)), ((), ())),
                                preferred_element_type=jnp.float32)
        out_ref[i] = o + b2_ref[i]


def _blockdiag(a):
    heads, oc = a.shape
    eye = jnp.eye(heads, dtype=a.dtype)
    return (a[:, :, None] * eye[:, None, :]).reshape(heads * oc, heads)


def kernel(x, edge_index, W1, a_s1, a_d1, b1, W2, a_s2, a_d2, b2,
           fW1, fb1, fW2, fb2):
    flat = edge_index[1] * NN + edge_index[0]
    diag = jnp.arange(NN, dtype=jnp.int32) * (NN + 1)
    pad = jnp.full((_SC_NIDX - E - NN,), _SC_PAD_IDX, dtype=jnp.int32)
    idx = jnp.concatenate([flat, diag, pad])
    c_flat = _make_count_sc()(idx, jnp.zeros((_SC_TOTAL,), jnp.float32),
                              jnp.ones((_SC_NIDX,), jnp.float32))
    cmat = c_flat[:NN * NN].reshape(NN, NN)

    full = lambda shp: pl.BlockSpec(shp, lambda b: (0,) * len(shp))
    gat = pl.pallas_call(
        _gat_body,
        grid=(B,),
        in_specs=[
            pl.BlockSpec((1, SEQ, NN), lambda b: (b, 0, 0)),
            full((SEQ, HEADS * H1)),
            full((HEADS * H1, HEADS)), full((HEADS * H1, HEADS)),
            full((1, H1)),
            full((H1, HEADS * H2)),
            full((HEADS * H2, HEADS)), full((HEADS * H2, HEADS)),
            full((1, H2)),
            full((NN, NN)),
        ],
        out_specs=pl.BlockSpec((1, NN, H2), lambda b: (b, 0, 0)),
        out_shape=jax.ShapeDtypeStruct((B, NN, H2), jnp.float32),
    )(x, W1, _blockdiag(a_s1), _blockdiag(a_d1), b1.reshape(1, H1),
      W2, _blockdiag(a_s2), _blockdiag(a_d2), b2.reshape(1, H2), cmat)

    pred = pl.pallas_call(
        _mlp_body,
        grid=((NN + NT - 1) // NT,),
        in_specs=[
            pl.BlockSpec((B, NT, H2), lambda n: (0, n, 0)),
            pl.BlockSpec((NT, H2, 64), lambda n: (n, 0, 0)),
            pl.BlockSpec((NT, 1, 64), lambda n: (n, 0, 0)),
            pl.BlockSpec((NT, 64, OUT), lambda n: (n, 0, 0)),
            pl.BlockSpec((NT, 1, OUT), lambda n: (n, 0, 0)),
        ],
        out_specs=pl.BlockSpec((NT, B, OUT), lambda n: (n, 0, 0)),
        out_shape=jax.ShapeDtypeStruct((NN, B, OUT), jnp.float32),
    )(gat, fW1, fb1.reshape(NN, 1, 64), fW2, fb2.reshape(NN, 1, OUT))
    return pred


# speedup vs baseline: 1.0011x; 1.0011x over previous
import functools

import jax
import jax.numpy as jnp
from jax.experimental import pallas as pl
from jax.experimental.pallas import tpu as pltpu
from jax.experimental.pallas import tpu_sc as plsc

B = 64
SEQ = 96
NN = 325
E = 2600
OUT = 24
HEADS = 4
H1 = 64
H2 = 128
NT = 8


_SC_NW = 16
_SC_CHUNK = 184
_SC_NIDX = _SC_NW * _SC_CHUNK
_SC_TOTAL = 105728
_SC_ZCHUNK = _SC_TOTAL // _SC_NW
_SC_PAD_IDX = 105626


def _make_count_sc():
    mesh = plsc.VectorSubcoreMesh(core_axis_name="c", subcore_axis_name="s",
                                  num_cores=1)

    @functools.partial(
        pl.kernel, mesh=mesh,
        out_type=jax.ShapeDtypeStruct((_SC_TOTAL,), jnp.float32),
        scratch_types=[
            pltpu.VMEM((_SC_CHUNK,), jnp.int32),
            pltpu.VMEM((_SC_CHUNK,), jnp.float32),
            pltpu.VMEM((_SC_ZCHUNK,), jnp.float32),
            pltpu.VMEM_SHARED((_SC_TOTAL,), jnp.float32),
        ],
    )
    def count_sc(idx_hbm, ones_hbm, out_hbm, idx_v, ones_v,
                 zbuf_v, shared):
        wid = jax.lax.axis_index("s")
        zb = wid * _SC_ZCHUNK
        zbuf_v[...] = jnp.zeros((_SC_ZCHUNK,), jnp.float32)
        pltpu.sync_copy(zbuf_v, shared.at[pl.ds(zb, _SC_ZCHUNK)])
        plsc.subcore_barrier()
        eb = wid * _SC_CHUNK
        pltpu.sync_copy(idx_hbm.at[pl.ds(eb, _SC_CHUNK)], idx_v)
        pltpu.sync_copy(ones_hbm.at[pl.ds(eb, _SC_CHUNK)], ones_v)
        pltpu.sync_copy(ones_v, shared.at[idx_v], add=True)
        plsc.subcore_barrier()
        pltpu.sync_copy(shared.at[pl.ds(zb, _SC_ZCHUNK)], zbuf_v)
        pltpu.sync_copy(zbuf_v, out_hbm.at[pl.ds(zb, _SC_ZCHUNK)])

    return count_sc


def _gat_layer(h, as_bd, ad_bd, cmat, oc):
    als_t = jax.lax.dot_general(as_bd, h, (((0,), (1,)), ((), ())),
                                preferred_element_type=jnp.float32)
    ald = jax.lax.dot_general(h, ad_bd, (((1,), (0,)), ((), ())),
                              preferred_element_type=jnp.float32)
    ones_col = jnp.ones((NN, 1), dtype=jnp.float32)
    ones_row = jnp.ones((1, NN), dtype=jnp.float32)
    acc = jnp.zeros((NN, oc), dtype=jnp.float32)
    for k in range(HEADS):
        row = als_t[k:k + 1, :]
        col = ald[:, k:k + 1]
        t = col + jnp.max(row)
        bd = jnp.maximum(t, 0.2 * t)
        e = col + row
        e = jnp.maximum(e, 0.2 * e)
        p = cmat * jnp.exp(e - bd)
        den = p.sum(axis=1, keepdims=True)
        rden = 1.0 / (den + 1e-16)
        hh = h[:, k * oc:(k + 1) * oc]
        num = jax.lax.dot_general(p, hh, (((1,), (0,)), ((), ())),
                                  preferred_element_type=jnp.float32)
        acc = acc + num * rden
    return acc * (1.0 / HEADS)


def _gat_body(x_ref, w1_ref, as1_ref, ad1_ref, b1_ref, w2_ref, as2_ref,
              ad2_ref, b2_ref, c_ref, out_ref):
    xb = x_ref[0]
    cmat = c_ref[...]

    h = jax.lax.dot_general(xb, w1_ref[...], (((0,), (0,)), ((), ())),
                            preferred_element_type=jnp.float32)
    o1 = _gat_layer(h, as1_ref[...], ad1_ref[...], cmat, H1) + b1_ref[...]
    o1 = jnp.where(o1 > 0, o1, jnp.exp(jnp.minimum(o1, 0.0)) - 1.0)

    h = jax.lax.dot_general(o1, w2_ref[...], (((1,), (0,)), ((), ())),
                            preferred_element_type=jnp.float32)
    out_ref[0] = _gat_layer(h, as2_ref[...], ad2_ref[...], cmat, H2) \
        + b2_ref[...]


def _mlp_body(g_ref, w1_ref, b1_ref, w2_ref, b2_ref, out_ref):
    for i in range(NT):
        g = g_ref[:, i, :]
        t = jax.lax.dot_general(g, w1_ref[i], (((1,), (0,)), ((), ())),
                                preferred_element_type=jnp.float32)
        t = jnp.maximum(t + b1_ref[i], 0.0)
        o = jax.lax.dot_general(t, w2_ref[i], (((1,), (0,)), ((), ())),
                                preferred_element_type=jnp.float32)
        out_ref[i] = o + b2_ref[i]


def _blockdiag(a):
    heads, oc = a.shape
    eye = jnp.eye(heads, dtype=a.dtype)
    return (a[:, :, None] * eye[:, None, :]).reshape(heads * oc, heads)


def kernel(x, edge_index, W1, a_s1, a_d1, b1, W2, a_s2, a_d2, b2,
           fW1, fb1, fW2, fb2):
    flat = edge_index[1] * NN + edge_index[0]
    diag = jnp.arange(NN, dtype=jnp.int32) * (NN + 1)
    pad = jnp.full((_SC_NIDX - E - NN,), _SC_PAD_IDX, dtype=jnp.int32)
    idx = jnp.concatenate([flat, diag, pad])
    c_flat = _make_count_sc()(idx, jnp.ones((_SC_NIDX,), jnp.float32))
    cmat = c_flat[:NN * NN].reshape(NN, NN)

    full = lambda shp: pl.BlockSpec(shp, lambda b: (0,) * len(shp))
    gat = pl.pallas_call(
        _gat_body,
        grid=(B,),
        in_specs=[
            pl.BlockSpec((1, SEQ, NN), lambda b: (b, 0, 0)),
            full((SEQ, HEADS * H1)),
            full((HEADS * H1, HEADS)), full((HEADS * H1, HEADS)),
            full((1, H1)),
            full((H1, HEADS * H2)),
            full((HEADS * H2, HEADS)), full((HEADS * H2, HEADS)),
            full((1, H2)),
            full((NN, NN)),
        ],
        out_specs=pl.BlockSpec((1, NN, H2), lambda b: (b, 0, 0)),
        out_shape=jax.ShapeDtypeStruct((B, NN, H2), jnp.float32),
    )(x, W1, _blockdiag(a_s1), _blockdiag(a_d1), b1.reshape(1, H1),
      W2, _blockdiag(a_s2), _blockdiag(a_d2), b2.reshape(1, H2), cmat)

    pred = pl.pallas_call(
        _mlp_body,
        grid=((NN + NT - 1) // NT,),
        in_specs=[
            pl.BlockSpec((B, NT, H2), lambda n: (0, n, 0)),
            pl.BlockSpec((NT, H2, 64), lambda n: (n, 0, 0)),
            pl.BlockSpec((NT, 1, 64), lambda n: (n, 0, 0)),
            pl.BlockSpec((NT, 64, OUT), lambda n: (n, 0, 0)),
            pl.BlockSpec((NT, 1, OUT), lambda n: (n, 0, 0)),
        ],
        out_specs=pl.BlockSpec((NT, B, OUT), lambda n: (n, 0, 0)),
        out_shape=jax.ShapeDtypeStruct((NN, B, OUT), jnp.float32),
    )(gat, fW1, fb1.reshape(NN, 1, 64), fW2, fb2.reshape(NN, 1, OUT))
    return pred


# speedup vs baseline: 1.0349x; 1.0337x over previous
import functools

import jax
import jax.numpy as jnp
from jax.experimental import pallas as pl
from jax.experimental.pallas import tpu as pltpu
from jax.experimental.pallas import tpu_sc as plsc

B = 64
SEQ = 96
NN = 325
E = 2600
OUT = 24
HEADS = 4
H1 = 64
H2 = 128
NT = 8
BT = 2


_SC_NW = 16
_SC_CHUNK = 184
_SC_NIDX = _SC_NW * _SC_CHUNK
_SC_TOTAL = 105728
_SC_ZCHUNK = _SC_TOTAL // _SC_NW
_SC_PAD_IDX = 105626


def _make_count_sc():
    mesh = plsc.VectorSubcoreMesh(core_axis_name="c", subcore_axis_name="s",
                                  num_cores=1)

    @functools.partial(
        pl.kernel, mesh=mesh,
        out_type=jax.ShapeDtypeStruct((_SC_TOTAL,), jnp.float32),
        scratch_types=[
            pltpu.VMEM((_SC_CHUNK,), jnp.int32),
            pltpu.VMEM((_SC_CHUNK,), jnp.float32),
            pltpu.VMEM((_SC_ZCHUNK,), jnp.float32),
            pltpu.VMEM_SHARED((_SC_TOTAL,), jnp.float32),
        ],
    )
    def count_sc(idx_hbm, ones_hbm, out_hbm, idx_v, ones_v,
                 zbuf_v, shared):
        wid = jax.lax.axis_index("s")
        zb = wid * _SC_ZCHUNK
        zbuf_v[...] = jnp.zeros((_SC_ZCHUNK,), jnp.float32)
        pltpu.sync_copy(zbuf_v, shared.at[pl.ds(zb, _SC_ZCHUNK)])
        plsc.subcore_barrier()
        eb = wid * _SC_CHUNK
        pltpu.sync_copy(idx_hbm.at[pl.ds(eb, _SC_CHUNK)], idx_v)
        pltpu.sync_copy(ones_hbm.at[pl.ds(eb, _SC_CHUNK)], ones_v)
        pltpu.sync_copy(ones_v, shared.at[idx_v], add=True)
        plsc.subcore_barrier()
        pltpu.sync_copy(shared.at[pl.ds(zb, _SC_ZCHUNK)], zbuf_v)
        pltpu.sync_copy(zbuf_v, out_hbm.at[pl.ds(zb, _SC_ZCHUNK)])

    return count_sc


def _gat_layer(h, as_bd, ad_bd, cmat, oc):
    als_t = jax.lax.dot_general(as_bd, h, (((0,), (1,)), ((), ())),
                                preferred_element_type=jnp.float32)
    ald = jax.lax.dot_general(h, ad_bd, (((1,), (0,)), ((), ())),
                              preferred_element_type=jnp.float32)
    ones_col = jnp.ones((NN, 1), dtype=jnp.float32)
    ones_row = jnp.ones((1, NN), dtype=jnp.float32)
    acc = jnp.zeros((NN, oc), dtype=jnp.float32)
    for k in range(HEADS):
        row = als_t[k:k + 1, :]
        col = ald[:, k:k + 1]
        t = col + jnp.max(row)
        bd = jnp.maximum(t, 0.2 * t)
        e = col + row
        e = jnp.maximum(e, 0.2 * e)
        p = cmat * jnp.exp(e - bd)
        den = p.sum(axis=1, keepdims=True)
        rden = 1.0 / (den + 1e-16)
        hh = h[:, k * oc:(k + 1) * oc]
        num = jax.lax.dot_general(p, hh, (((1,), (0,)), ((), ())),
                                  preferred_element_type=jnp.float32)
        acc = acc + num * rden
    return acc * (1.0 / HEADS)


def _gat_body(x_ref, w1_ref, as1_ref, ad1_ref, b1_ref, w2_ref, as2_ref,
              ad2_ref, b2_ref, c_ref, out_ref):
    cmat = c_ref[...]
    for j in range(BT):
        xb = x_ref[j]
        h = jax.lax.dot_general(xb, w1_ref[...], (((0,), (0,)), ((), ())),
                                preferred_element_type=jnp.float32)
        o1 = _gat_layer(h, as1_ref[...], ad1_ref[...], cmat, H1) \
            + b1_ref[...]
        o1 = jnp.where(o1 > 0, o1,
                       jnp.exp(jnp.minimum(o1, 0.0)) - 1.0)
        h = jax.lax.dot_general(o1, w2_ref[...], (((1,), (0,)), ((), ())),
                                preferred_element_type=jnp.float32)
        out_ref[j] = _gat_layer(h, as2_ref[...], ad2_ref[...], cmat, H2) \
            + b2_ref[...]


def _mlp_body(g_ref, w1_ref, b1_ref, w2_ref, b2_ref, out_ref):
    for i in range(NT):
        g = g_ref[:, i, :]
        t = jax.lax.dot_general(g, w1_ref[i], (((1,), (0,)), ((), ())),
                                preferred_element_type=jnp.float32)
        t = jnp.maximum(t + b1_ref[i], 0.0)
        o = jax.lax.dot_general(t, w2_ref[i], (((1,), (0,)), ((), ())),
                                preferred_element_type=jnp.float32)
        out_ref[i] = o + b2_ref[i]


def _blockdiag(a):
    heads, oc = a.shape
    eye = jnp.eye(heads, dtype=a.dtype)
    return (a[:, :, None] * eye[:, None, :]).reshape(heads * oc, heads)


def kernel(x, edge_index, W1, a_s1, a_d1, b1, W2, a_s2, a_d2, b2,
           fW1, fb1, fW2, fb2):
    flat = edge_index[1] * NN + edge_index[0]
    diag = jnp.arange(NN, dtype=jnp.int32) * (NN + 1)
    pad = jnp.full((_SC_NIDX - E - NN,), _SC_PAD_IDX, dtype=jnp.int32)
    idx = jnp.concatenate([flat, diag, pad])
    c_flat = _make_count_sc()(idx, jnp.ones((_SC_NIDX,), jnp.float32))
    cmat = c_flat[:NN * NN].reshape(NN, NN)

    full = lambda shp: pl.BlockSpec(shp, lambda b: (0,) * len(shp))
    gat = pl.pallas_call(
        _gat_body,
        grid=(B // BT,),
        in_specs=[
            pl.BlockSpec((BT, SEQ, NN), lambda b: (b, 0, 0)),
            full((SEQ, HEADS * H1)),
            full((HEADS * H1, HEADS)), full((HEADS * H1, HEADS)),
            full((1, H1)),
            full((H1, HEADS * H2)),
            full((HEADS * H2, HEADS)), full((HEADS * H2, HEADS)),
            full((1, H2)),
            full((NN, NN)),
        ],
        out_specs=pl.BlockSpec((BT, NN, H2), lambda b: (b, 0, 0)),
        out_shape=jax.ShapeDtypeStruct((B, NN, H2), jnp.float32),
    )(x, W1, _blockdiag(a_s1), _blockdiag(a_d1), b1.reshape(1, H1),
      W2, _blockdiag(a_s2), _blockdiag(a_d2), b2.reshape(1, H2), cmat)

    pred = pl.pallas_call(
        _mlp_body,
        grid=((NN + NT - 1) // NT,),
        in_specs=[
            pl.BlockSpec((B, NT, H2), lambda n: (0, n, 0)),
            pl.BlockSpec((NT, H2, 64), lambda n: (n, 0, 0)),
            pl.BlockSpec((NT, 1, 64), lambda n: (n, 0, 0)),
            pl.BlockSpec((NT, 64, OUT), lambda n: (n, 0, 0)),
            pl.BlockSpec((NT, 1, OUT), lambda n: (n, 0, 0)),
        ],
        out_specs=pl.BlockSpec((NT, B, OUT), lambda n: (n, 0, 0)),
        out_shape=jax.ShapeDtypeStruct((NN, B, OUT), jnp.float32),
    )(gat, fW1, fb1.reshape(NN, 1, 64), fW2, fb2.reshape(NN, 1, OUT))
    return pred


# speedup vs baseline: 1.0475x; 1.0123x over previous
import functools

import jax
import jax.numpy as jnp
from jax.experimental import pallas as pl
from jax.experimental.pallas import tpu as pltpu
from jax.experimental.pallas import tpu_sc as plsc

B = 64
SEQ = 96
NN = 325
E = 2600
OUT = 24
HEADS = 4
H1 = 64
H2 = 128
NT = 16
BT = 4


_SC_NW = 16
_SC_CHUNK = 184
_SC_NIDX = _SC_NW * _SC_CHUNK
_SC_TOTAL = 105728
_SC_ZCHUNK = _SC_TOTAL // _SC_NW
_SC_PAD_IDX = 105626


def _make_count_sc():
    mesh = plsc.VectorSubcoreMesh(core_axis_name="c", subcore_axis_name="s",
                                  num_cores=1)

    @functools.partial(
        pl.kernel, mesh=mesh,
        out_type=jax.ShapeDtypeStruct((_SC_TOTAL,), jnp.float32),
        scratch_types=[
            pltpu.VMEM((_SC_CHUNK,), jnp.int32),
            pltpu.VMEM((_SC_CHUNK,), jnp.float32),
            pltpu.VMEM((_SC_ZCHUNK,), jnp.float32),
            pltpu.VMEM_SHARED((_SC_TOTAL,), jnp.float32),
        ],
    )
    def count_sc(idx_hbm, ones_hbm, out_hbm, idx_v, ones_v,
                 zbuf_v, shared):
        wid = jax.lax.axis_index("s")
        zb = wid * _SC_ZCHUNK
        zbuf_v[...] = jnp.zeros((_SC_ZCHUNK,), jnp.float32)
        pltpu.sync_copy(zbuf_v, shared.at[pl.ds(zb, _SC_ZCHUNK)])
        plsc.subcore_barrier()
        eb = wid * _SC_CHUNK
        pltpu.sync_copy(idx_hbm.at[pl.ds(eb, _SC_CHUNK)], idx_v)
        pltpu.sync_copy(ones_hbm.at[pl.ds(eb, _SC_CHUNK)], ones_v)
        pltpu.sync_copy(ones_v, shared.at[idx_v], add=True)
        plsc.subcore_barrier()
        pltpu.sync_copy(shared.at[pl.ds(zb, _SC_ZCHUNK)], zbuf_v)
        pltpu.sync_copy(zbuf_v, out_hbm.at[pl.ds(zb, _SC_ZCHUNK)])

    return count_sc


def _gat_layer(h, as_bd, ad_bd, cmat, oc):
    als_t = jax.lax.dot_general(as_bd, h, (((0,), (1,)), ((), ())),
                                preferred_element_type=jnp.float32)
    ald = jax.lax.dot_general(h, ad_bd, (((1,), (0,)), ((), ())),
                              preferred_element_type=jnp.float32)
    ones_col = jnp.ones((NN, 1), dtype=jnp.float32)
    ones_row = jnp.ones((1, NN), dtype=jnp.float32)
    acc = jnp.zeros((NN, oc), dtype=jnp.float32)
    for k in range(HEADS):
        row = als_t[k:k + 1, :]
        col = ald[:, k:k + 1]
        t = col + jnp.max(row)
        bd = jnp.maximum(t, 0.2 * t)
        e = col + row
        e = jnp.maximum(e, 0.2 * e)
        p = cmat * jnp.exp(e - bd)
        den = p.sum(axis=1, keepdims=True)
        rden = 1.0 / (den + 1e-16)
        hh = h[:, k * oc:(k + 1) * oc]
        num = jax.lax.dot_general(p, hh, (((1,), (0,)), ((), ())),
                                  preferred_element_type=jnp.float32)
        acc = acc + num * rden
    return acc * (1.0 / HEADS)


def _gat_body(x_ref, w1_ref, as1_ref, ad1_ref, b1_ref, w2_ref, as2_ref,
              ad2_ref, b2_ref, c_ref, out_ref):
    cmat = c_ref[...]
    for j in range(BT):
        xb = x_ref[j]
        h = jax.lax.dot_general(xb, w1_ref[...], (((0,), (0,)), ((), ())),
                                preferred_element_type=jnp.float32)
        o1 = _gat_layer(h, as1_ref[...], ad1_ref[...], cmat, H1) \
            + b1_ref[...]
        o1 = jnp.where(o1 > 0, o1,
                       jnp.exp(jnp.minimum(o1, 0.0)) - 1.0)
        h = jax.lax.dot_general(o1, w2_ref[...], (((1,), (0,)), ((), ())),
                                preferred_element_type=jnp.float32)
        out_ref[j] = _gat_layer(h, as2_ref[...], ad2_ref[...], cmat, H2) \
            + b2_ref[...]


def _mlp_body(g_ref, w1_ref, b1_ref, w2_ref, b2_ref, out_ref):
    for i in range(NT):
        g = g_ref[:, i, :]
        t = jax.lax.dot_general(g, w1_ref[i], (((1,), (0,)), ((), ())),
                                preferred_element_type=jnp.float32)
        t = jnp.maximum(t + b1_ref[i], 0.0)
        o = jax.lax.dot_general(t, w2_ref[i], (((1,), (0,)), ((), ())),
                                preferred_element_type=jnp.float32)
        out_ref[i] = o + b2_ref[i]


def _blockdiag(a):
    heads, oc = a.shape
    eye = jnp.eye(heads, dtype=a.dtype)
    return (a[:, :, None] * eye[:, None, :]).reshape(heads * oc, heads)


def kernel(x, edge_index, W1, a_s1, a_d1, b1, W2, a_s2, a_d2, b2,
           fW1, fb1, fW2, fb2):
    flat = edge_index[1] * NN + edge_index[0]
    diag = jnp.arange(NN, dtype=jnp.int32) * (NN + 1)
    pad = jnp.full((_SC_NIDX - E - NN,), _SC_PAD_IDX, dtype=jnp.int32)
    idx = jnp.concatenate([flat, diag, pad])
    c_flat = _make_count_sc()(idx, jnp.ones((_SC_NIDX,), jnp.float32))
    cmat = c_flat[:NN * NN].reshape(NN, NN)

    full = lambda shp: pl.BlockSpec(shp, lambda b: (0,) * len(shp))
    gat = pl.pallas_call(
        _gat_body,
        grid=(B // BT,),
        in_specs=[
            pl.BlockSpec((BT, SEQ, NN), lambda b: (b, 0, 0)),
            full((SEQ, HEADS * H1)),
            full((HEADS * H1, HEADS)), full((HEADS * H1, HEADS)),
            full((1, H1)),
            full((H1, HEADS * H2)),
            full((HEADS * H2, HEADS)), full((HEADS * H2, HEADS)),
            full((1, H2)),
            full((NN, NN)),
        ],
        out_specs=pl.BlockSpec((BT, NN, H2), lambda b: (b, 0, 0)),
        out_shape=jax.ShapeDtypeStruct((B, NN, H2), jnp.float32),
    )(x, W1, _blockdiag(a_s1), _blockdiag(a_d1), b1.reshape(1, H1),
      W2, _blockdiag(a_s2), _blockdiag(a_d2), b2.reshape(1, H2), cmat)

    pred = pl.pallas_call(
        _mlp_body,
        grid=((NN + NT - 1) // NT,),
        in_specs=[
            pl.BlockSpec((B, NT, H2), lambda n: (0, n, 0)),
            pl.BlockSpec((NT, H2, 64), lambda n: (n, 0, 0)),
            pl.BlockSpec((NT, 1, 64), lambda n: (n, 0, 0)),
            pl.BlockSpec((NT, 64, OUT), lambda n: (n, 0, 0)),
            pl.BlockSpec((NT, 1, OUT), lambda n: (n, 0, 0)),
        ],
        out_specs=pl.BlockSpec((NT, B, OUT), lambda n: (n, 0, 0)),
        out_shape=jax.ShapeDtypeStruct((NN, B, OUT), jnp.float32),
    )(gat, fW1, fb1.reshape(NN, 1, 64), fW2, fb2.reshape(NN, 1, OUT))
    return pred
